# Initial kernel scaffold; baseline (speedup 1.0000x reference)
import jax
import jax.numpy as jnp
from jax.experimental import pallas as pl


def kernel(x, expert_weights, expert_indices, batch_size_per_expert, w1, w2):
    raise NotImplementedError("write your pallas kernel here")


# speedup vs baseline: 8.0862x; 8.0862x over previous
import functools

import jax
import jax.numpy as jnp
from jax import lax
from jax.experimental import pallas as pl
from jax.experimental.pallas import tpu as pltpu
from jax.experimental.pallas import tpu_sc as plsc

E = 64
TOP_K = 8
D_MODEL = 1024
D_FF = 512
N_TOKENS = 32768
P = N_TOKENS * TOP_K
BLK = 256
NB = P // BLK + E
PP = NB * BLK

NUM_WORKERS = 32


def _routing_tables(expert_weights, expert_indices, batch_size_per_expert):
    flat_idx = expert_indices.reshape(-1).astype(jnp.int32)
    flat_w = expert_weights.reshape(-1)
    counts = batch_size_per_expert.astype(jnp.int32)

    order = jnp.argsort(flat_idx, stable=True)
    csum = jnp.cumsum(counts)
    starts = csum - counts
    blocks_per_e = (counts + BLK - 1) // BLK
    bcum = jnp.cumsum(blocks_per_e)
    block_start = bcum - blocks_per_e
    total_blocks = bcum[-1]
    padded_start = block_start * BLK

    block_expert = jnp.searchsorted(bcum, jnp.arange(NB, dtype=jnp.int32),
                                    side="right").astype(jnp.int32)
    block_expert = jnp.minimum(block_expert, E - 1)

    p = jnp.arange(PP, dtype=jnp.int32)
    e_p = block_expert[p // BLK]
    off = p - padded_start[e_p]
    j = starts[e_p] + off
    valid = (off >= 0) & (off < counts[e_p])
    a = order[jnp.clip(j, 0, P - 1)]
    src_tok = jnp.where(valid, a // TOP_K, 0).astype(jnp.int32)
    slot_w = jnp.where(valid, flat_w[a], 0.0).astype(jnp.float32)

    sorted_e = flat_idx[order]
    slotpos = (padded_start[sorted_e]
               + jnp.arange(P, dtype=jnp.int32) - starts[sorted_e])
    slot_of_assign = jnp.zeros((P,), jnp.int32).at[order].set(slotpos)

    return block_expert, src_tok, slot_w, slot_of_assign


_G_CHUNK = 64
_G_ROWS_PER_W = PP // NUM_WORKERS
_G_NCHUNK = _G_ROWS_PER_W // _G_CHUNK


def _make_gather():
    mesh = plsc.VectorSubcoreMesh(core_axis_name="c", subcore_axis_name="s")

    @functools.partial(
        pl.kernel, mesh=mesh,
        out_type=jax.ShapeDtypeStruct((PP, D_MODEL), jnp.float32),
        scratch_types=[
            pltpu.VMEM((_G_CHUNK,), jnp.int32),
            pltpu.VMEM((_G_CHUNK, D_MODEL), jnp.float32),
            pltpu.SemaphoreType.DMA,
        ],
    )
    def gather_k(x_hbm, idx_hbm, out_hbm, idx_v, rows_v, sem):
        wid = lax.axis_index("s") * 2 + lax.axis_index("c")
        base = wid * _G_ROWS_PER_W

        def body(c, carry):
            off = base + c * _G_CHUNK
            pltpu.sync_copy(idx_hbm.at[pl.ds(off, _G_CHUNK)], idx_v)
            pltpu.async_copy(x_hbm.at[idx_v], rows_v, sem).wait()
            pltpu.sync_copy(rows_v, out_hbm.at[pl.ds(off, _G_CHUNK)])
            return carry

        lax.fori_loop(0, _G_NCHUNK, body, 0)

    return gather_k


def _mlp_block(be_ref, xb_ref, w1_ref, w2_ref, sw_ref, ob_ref):
    del be_ref
    xb = xb_ref[...]
    h = jax.nn.gelu(jnp.dot(xb, w1_ref[0], preferred_element_type=jnp.float32))
    o = jnp.dot(h, w2_ref[0], preferred_element_type=jnp.float32)
    ob_ref[...] = o * sw_ref[0, 0][:, None]


def _grouped_mlp(block_expert, x_perm, w1, w2, slot_w):
    grid_spec = pltpu.PrefetchScalarGridSpec(
        num_scalar_prefetch=1,
        grid=(NB,),
        in_specs=[
            pl.BlockSpec((BLK, D_MODEL), lambda b, be: (b, 0)),
            pl.BlockSpec((1, D_MODEL, D_FF), lambda b, be: (be[b], 0, 0)),
            pl.BlockSpec((1, D_FF, D_MODEL), lambda b, be: (be[b], 0, 0)),
            pl.BlockSpec((1, 1, BLK), lambda b, be: (b, 0, 0)),
        ],
        out_specs=pl.BlockSpec((BLK, D_MODEL), lambda b, be: (b, 0)),
    )
    return pl.pallas_call(
        _mlp_block,
        grid_spec=grid_spec,
        out_shape=jax.ShapeDtypeStruct((PP, D_MODEL), jnp.float32),
    )(block_expert, x_perm, w1, w2, slot_w.reshape(NB, 1, BLK))


_U_TOK = 8
_U_TOK_PER_W = N_TOKENS // NUM_WORKERS
_U_NCHUNK = _U_TOK_PER_W // _U_TOK
_NV = D_MODEL // 16


def _make_unpermute():
    mesh = plsc.VectorSubcoreMesh(core_axis_name="c", subcore_axis_name="s")

    @functools.partial(
        pl.kernel, mesh=mesh,
        out_type=jax.ShapeDtypeStruct((N_TOKENS, D_MODEL), jnp.float32),
        scratch_types=[
            pltpu.VMEM((_U_TOK * TOP_K,), jnp.int32),
            pltpu.VMEM((_U_TOK * TOP_K, D_MODEL), jnp.float32),
            pltpu.VMEM((_U_TOK, D_MODEL), jnp.float32),
            pltpu.SemaphoreType.DMA,
        ],
    )
    def unperm_k(outp_hbm, sidx_hbm, y_hbm, idx_v, rows_v, y_v, sem):
        wid = lax.axis_index("s") * 2 + lax.axis_index("c")
        tok_base = wid * _U_TOK_PER_W

        def chunk_body(c, carry):
            t0 = tok_base + c * _U_TOK
            pltpu.sync_copy(sidx_hbm.at[pl.ds(t0 * TOP_K, _U_TOK * TOP_K)],
                            idx_v)
            pltpu.async_copy(outp_hbm.at[idx_v], rows_v, sem).wait()

            def col_body(v, carry2):
                sl = pl.ds(v * 16, 16)
                for j in range(_U_TOK):
                    acc = rows_v[j * TOP_K, sl]
                    for r in range(1, TOP_K):
                        acc = acc + rows_v[j * TOP_K + r, sl]
                    y_v[j, sl] = acc
                return carry2

            lax.fori_loop(0, _NV, col_body, 0)
            pltpu.sync_copy(y_v, y_hbm.at[pl.ds(t0, _U_TOK)])
            return carry

        lax.fori_loop(0, _U_NCHUNK, chunk_body, 0)

    return unperm_k


def kernel(x, expert_weights, expert_indices, batch_size_per_expert, w1, w2):
    block_expert, src_tok, slot_w, slot_of_assign = _routing_tables(
        expert_weights, expert_indices, batch_size_per_expert)

    x_perm = _make_gather()(x, src_tok)
    out_perm = _grouped_mlp(block_expert, x_perm, w1, w2, slot_w)
    y = _make_unpermute()(out_perm, slot_of_assign)
    return y


# speedup vs baseline: 12.5591x; 1.5532x over previous
import functools

import jax
import jax.numpy as jnp
from jax import lax
from jax.experimental import pallas as pl
from jax.experimental.pallas import tpu as pltpu
from jax.experimental.pallas import tpu_sc as plsc

E = 64
TOP_K = 8
D_MODEL = 1024
D_FF = 512
N_TOKENS = 32768
P = N_TOKENS * TOP_K
BLK = 256
NB = P // BLK + E
PP = NB * BLK

NUM_WORKERS = 32


_G = 2048
_g = P // _G


def _routing_tables(expert_indices, batch_size_per_expert):
    flat_idx = expert_indices.reshape(-1).astype(jnp.int32)
    counts = batch_size_per_expert.astype(jnp.int32)

    blocks_per_e = (counts + BLK - 1) // BLK
    bcum = jnp.cumsum(blocks_per_e)
    block_start = bcum - blocks_per_e
    padded_start = block_start * BLK

    block_expert = jnp.sum(
        jnp.arange(NB, dtype=jnp.int32)[:, None] >= bcum[None, :],
        axis=1, dtype=jnp.int32)
    block_expert = jnp.minimum(block_expert, E - 1)

    eg = flat_idx.reshape(_G, _g)
    ohg = (eg[:, :, None] == jnp.arange(E, dtype=jnp.int32)).astype(
        jnp.bfloat16)
    lg = (jnp.arange(_g)[:, None] > jnp.arange(_g)[None, :]).astype(
        jnp.bfloat16)
    intra = jnp.einsum("ij,gje->gie", lg, ohg,
                       preferred_element_type=jnp.float32)
    gsum = jnp.sum(ohg, axis=1, dtype=jnp.float32)
    lG = (jnp.arange(_G)[:, None] > jnp.arange(_G)[None, :]).astype(
        jnp.bfloat16)
    gbase = jnp.dot(lG, gsum.astype(jnp.bfloat16),
                    preferred_element_type=jnp.float32)

    t = padded_start.astype(jnp.float32)[None, :] + gbase
    dest = jnp.sum((t[:, None, :] + intra) * ohg.astype(jnp.float32),
                   axis=-1)
    dest = jnp.round(dest).astype(jnp.int32).reshape(-1)
    return block_expert, dest


_S_CHUNK = 128
_S_PER_W = P // NUM_WORKERS
_S_NCHUNK = _S_PER_W // _S_CHUNK


def _make_scatter_meta():
    mesh = plsc.VectorSubcoreMesh(core_axis_name="c", subcore_axis_name="s")

    @functools.partial(
        pl.kernel, mesh=mesh,
        out_type=(jax.ShapeDtypeStruct((PP,), jnp.int32),
                  jax.ShapeDtypeStruct((PP,), jnp.float32)),
        scratch_types=[
            pltpu.VMEM((_S_CHUNK,), jnp.int32),
            pltpu.VMEM((_S_CHUNK,), jnp.int32),
            pltpu.VMEM((_S_CHUNK,), jnp.float32),
            pltpu.SemaphoreType.DMA,
            pltpu.SemaphoreType.DMA,
        ],
    )
    def scatter_k(dest_hbm, tok_hbm, w_hbm, stok_hbm, sw_hbm,
                  dest_v, tok_v, w_v, sem1, sem2):
        wid = lax.axis_index("s") * 2 + lax.axis_index("c")
        base = wid * _S_PER_W

        def body(c, carry):
            off = base + c * _S_CHUNK
            pltpu.sync_copy(dest_hbm.at[pl.ds(off, _S_CHUNK)], dest_v)
            pltpu.sync_copy(tok_hbm.at[pl.ds(off, _S_CHUNK)], tok_v)
            pltpu.sync_copy(w_hbm.at[pl.ds(off, _S_CHUNK)], w_v)
            cp1 = pltpu.async_copy(tok_v, stok_hbm.at[dest_v], sem1)
            cp2 = pltpu.async_copy(w_v, sw_hbm.at[dest_v], sem2)
            cp1.wait()
            cp2.wait()
            return carry

        lax.fori_loop(0, _S_NCHUNK, body, 0)

    return scatter_k


_G_CHUNK = 64
_G_ROWS_PER_W = PP // NUM_WORKERS
_G_NCHUNK = _G_ROWS_PER_W // _G_CHUNK


def _make_gather():
    mesh = plsc.VectorSubcoreMesh(core_axis_name="c", subcore_axis_name="s")

    @functools.partial(
        pl.kernel, mesh=mesh,
        out_type=jax.ShapeDtypeStruct((PP, D_MODEL), jnp.float32),
        scratch_types=[
            pltpu.VMEM((_G_CHUNK,), jnp.int32),
            pltpu.VMEM((_G_CHUNK, D_MODEL), jnp.float32),
            pltpu.SemaphoreType.DMA,
        ],
    )
    def gather_k(x_hbm, idx_hbm, out_hbm, idx_v, rows_v, sem):
        wid = lax.axis_index("s") * 2 + lax.axis_index("c")
        base = wid * _G_ROWS_PER_W

        def body(c, carry):
            off = base + c * _G_CHUNK
            pltpu.sync_copy(idx_hbm.at[pl.ds(off, _G_CHUNK)], idx_v)
            for v in range(_G_CHUNK // 16):
                sl = pl.ds(v * 16, 16)
                idx_v[sl] = jnp.minimum(jnp.maximum(idx_v[sl], 0),
                                        N_TOKENS - 1)
            pltpu.async_copy(x_hbm.at[idx_v], rows_v, sem).wait()
            pltpu.sync_copy(rows_v, out_hbm.at[pl.ds(off, _G_CHUNK)])
            return carry

        lax.fori_loop(0, _G_NCHUNK, body, 0)

    return gather_k


def _mlp_block(be_ref, xb_ref, w1_ref, w2_ref, sw_ref, ob_ref):
    del be_ref
    xb = xb_ref[...]
    h = jax.nn.gelu(jnp.dot(xb, w1_ref[0], preferred_element_type=jnp.float32))
    o = jnp.dot(h, w2_ref[0], preferred_element_type=jnp.float32)
    ob_ref[...] = o * sw_ref[0, 0][:, None]


def _grouped_mlp(block_expert, x_perm, w1, w2, slot_w):
    grid_spec = pltpu.PrefetchScalarGridSpec(
        num_scalar_prefetch=1,
        grid=(NB,),
        in_specs=[
            pl.BlockSpec((BLK, D_MODEL), lambda b, be: (b, 0)),
            pl.BlockSpec((1, D_MODEL, D_FF), lambda b, be: (be[b], 0, 0)),
            pl.BlockSpec((1, D_FF, D_MODEL), lambda b, be: (be[b], 0, 0)),
            pl.BlockSpec((1, 1, BLK), lambda b, be: (b, 0, 0)),
        ],
        out_specs=pl.BlockSpec((BLK, D_MODEL), lambda b, be: (b, 0)),
    )
    return pl.pallas_call(
        _mlp_block,
        grid_spec=grid_spec,
        out_shape=jax.ShapeDtypeStruct((PP, D_MODEL), jnp.float32),
    )(block_expert, x_perm, w1, w2, slot_w.reshape(NB, 1, BLK))


_U_TOK = 8
_U_TOK_PER_W = N_TOKENS // NUM_WORKERS
_U_NCHUNK = _U_TOK_PER_W // _U_TOK
_NV = D_MODEL // 16


def _make_unpermute():
    mesh = plsc.VectorSubcoreMesh(core_axis_name="c", subcore_axis_name="s")

    @functools.partial(
        pl.kernel, mesh=mesh,
        out_type=jax.ShapeDtypeStruct((N_TOKENS, D_MODEL), jnp.float32),
        scratch_types=[
            pltpu.VMEM((_U_TOK * TOP_K,), jnp.int32),
            pltpu.VMEM((_U_TOK * TOP_K, D_MODEL), jnp.float32),
            pltpu.VMEM((_U_TOK, D_MODEL), jnp.float32),
            pltpu.SemaphoreType.DMA,
        ],
    )
    def unperm_k(outp_hbm, sidx_hbm, y_hbm, idx_v, rows_v, y_v, sem):
        wid = lax.axis_index("s") * 2 + lax.axis_index("c")
        tok_base = wid * _U_TOK_PER_W

        def chunk_body(c, carry):
            t0 = tok_base + c * _U_TOK
            pltpu.sync_copy(sidx_hbm.at[pl.ds(t0 * TOP_K, _U_TOK * TOP_K)],
                            idx_v)
            pltpu.async_copy(outp_hbm.at[idx_v], rows_v, sem).wait()

            def col_body(v, carry2):
                sl = pl.ds(v * 16, 16)
                for j in range(_U_TOK):
                    acc = rows_v[j * TOP_K, sl]
                    for r in range(1, TOP_K):
                        acc = acc + rows_v[j * TOP_K + r, sl]
                    y_v[j, sl] = acc
                return carry2

            lax.fori_loop(0, _NV, col_body, 0)
            pltpu.sync_copy(y_v, y_hbm.at[pl.ds(t0, _U_TOK)])
            return carry

        lax.fori_loop(0, _U_NCHUNK, chunk_body, 0)

    return unperm_k


def kernel(x, expert_weights, expert_indices, batch_size_per_expert, w1, w2):
    block_expert, dest = _routing_tables(expert_indices,
                                         batch_size_per_expert)
    flat_w = expert_weights.reshape(-1).astype(jnp.float32)
    tok_of_assign = jnp.arange(P, dtype=jnp.int32) // TOP_K

    src_tok, slot_w = _make_scatter_meta()(dest, tok_of_assign, flat_w)
    x_perm = _make_gather()(x, src_tok)
    out_perm = _grouped_mlp(block_expert, x_perm, w1, w2, slot_w)
    y = _make_unpermute()(out_perm, dest)
    return y


# speedup vs baseline: 13.3608x; 1.0638x over previous
import functools

import jax
import jax.numpy as jnp
from jax import lax
from jax.experimental import pallas as pl
from jax.experimental.pallas import tpu as pltpu
from jax.experimental.pallas import tpu_sc as plsc

E = 64
TOP_K = 8
D_MODEL = 1024
D_FF = 512
N_TOKENS = 32768
P = N_TOKENS * TOP_K
BLK = 256
NB = P // BLK + E
PP = NB * BLK

NUM_WORKERS = 32


_G = 2048
_g = P // _G


def _routing_tables(expert_indices, batch_size_per_expert):
    flat_idx = expert_indices.reshape(-1).astype(jnp.int32)
    counts = batch_size_per_expert.astype(jnp.int32)

    blocks_per_e = (counts + BLK - 1) // BLK
    bcum = jnp.cumsum(blocks_per_e)
    block_start = bcum - blocks_per_e
    padded_start = block_start * BLK

    block_expert = jnp.sum(
        jnp.arange(NB, dtype=jnp.int32)[:, None] >= bcum[None, :],
        axis=1, dtype=jnp.int32)
    block_expert = jnp.minimum(block_expert, E - 1)

    eg = flat_idx.reshape(_G, _g)
    ohg = (eg[:, :, None] == jnp.arange(E, dtype=jnp.int32)).astype(
        jnp.bfloat16)
    lg = (jnp.arange(_g)[:, None] > jnp.arange(_g)[None, :]).astype(
        jnp.bfloat16)
    intra = jnp.einsum("ij,gje->gie", lg, ohg,
                       preferred_element_type=jnp.float32)
    gsum = jnp.sum(ohg, axis=1, dtype=jnp.float32)
    lG = (jnp.arange(_G)[:, None] > jnp.arange(_G)[None, :]).astype(
        jnp.bfloat16)
    gbase = jnp.dot(lG, gsum.astype(jnp.bfloat16),
                    preferred_element_type=jnp.float32)

    t = padded_start.astype(jnp.float32)[None, :] + gbase
    dest = jnp.sum((t[:, None, :] + intra) * ohg.astype(jnp.float32),
                   axis=-1)
    dest = jnp.round(dest).astype(jnp.int32).reshape(-1)
    return block_expert, dest


_S_CHUNK = 128
_S_PER_W = P // NUM_WORKERS
_S_NCHUNK = _S_PER_W // _S_CHUNK
_S_WAVE = 8


def _make_scatter_meta():
    mesh = plsc.VectorSubcoreMesh(core_axis_name="c", subcore_axis_name="s")

    @functools.partial(
        pl.kernel, mesh=mesh,
        out_type=(jax.ShapeDtypeStruct((PP,), jnp.int32),
                  jax.ShapeDtypeStruct((PP,), jnp.float32)),
        scratch_types=[
            pltpu.VMEM((_S_NCHUNK, _S_CHUNK), jnp.int32),
            pltpu.VMEM((_S_NCHUNK, _S_CHUNK), jnp.int32),
            pltpu.VMEM((_S_NCHUNK, _S_CHUNK), jnp.float32),
            pltpu.SemaphoreType.DMA,
            pltpu.SemaphoreType.DMA,
            pltpu.SemaphoreType.DMA,
        ],
    )
    def scatter_k(dest_hbm, tok_hbm, w_hbm, stok_hbm, sw_hbm,
                  dest_v, tok_v, w_v, ps, sem1, sem2):
        wid = lax.axis_index("s") * 2 + lax.axis_index("c")
        rbase = wid * _S_NCHUNK

        pltpu.async_copy(dest_hbm.at[pl.ds(rbase, _S_NCHUNK)], dest_v, ps)
        pltpu.async_copy(tok_hbm.at[pl.ds(rbase, _S_NCHUNK)], tok_v, ps)
        pltpu.async_copy(w_hbm.at[pl.ds(rbase, _S_NCHUNK)], w_v, ps)
        pltpu.make_async_copy(dest_hbm.at[pl.ds(rbase, _S_NCHUNK)],
                              dest_v, ps).wait()
        pltpu.make_async_copy(tok_hbm.at[pl.ds(rbase, _S_NCHUNK)],
                              tok_v, ps).wait()
        pltpu.make_async_copy(w_hbm.at[pl.ds(rbase, _S_NCHUNK)],
                              w_v, ps).wait()

        def wave(wv, carry):
            for k in range(_S_WAVE):
                c = wv * _S_WAVE + k
                pltpu.async_copy(tok_v.at[c], stok_hbm.at[dest_v.at[c]],
                                 sem1)
                pltpu.async_copy(w_v.at[c], sw_hbm.at[dest_v.at[c]], sem2)
            for k in range(_S_WAVE):
                pltpu.make_async_copy(tok_v.at[0],
                                      stok_hbm.at[dest_v.at[0]],
                                      sem1).wait()
                pltpu.make_async_copy(w_v.at[0], sw_hbm.at[dest_v.at[0]],
                                      sem2).wait()
            return carry

        lax.fori_loop(0, _S_NCHUNK // _S_WAVE, wave, 0)

    return scatter_k


_G_CHUNK = 16
_G_NBUF = 4
_G_ROWS_PER_W = PP // NUM_WORKERS
_G_NCHUNK = _G_ROWS_PER_W // _G_CHUNK
_G_NGRP = _G_NCHUNK // _G_NBUF


def _make_gather():
    mesh = plsc.VectorSubcoreMesh(core_axis_name="c", subcore_axis_name="s")

    @functools.partial(
        pl.kernel, mesh=mesh,
        out_type=jax.ShapeDtypeStruct((PP, D_MODEL), jnp.float32),
        scratch_types=[
            pltpu.VMEM((_G_ROWS_PER_W,), jnp.int32),
            pltpu.VMEM((_G_NBUF, _G_CHUNK, D_MODEL), jnp.float32),
            pltpu.SemaphoreType.DMA,
            pltpu.SemaphoreType.DMA,
            pltpu.SemaphoreType.DMA,
            pltpu.SemaphoreType.DMA,
            pltpu.SemaphoreType.DMA,
            pltpu.SemaphoreType.DMA,
            pltpu.SemaphoreType.DMA,
            pltpu.SemaphoreType.DMA,
        ],
    )
    def gather_k(x_hbm, idx_hbm, out_hbm, idx_all, bufs,
                 gs0, gs1, gs2, gs3, ws0, ws1, ws2, ws3):
        wid = lax.axis_index("s") * 2 + lax.axis_index("c")
        base = wid * _G_ROWS_PER_W
        gs = (gs0, gs1, gs2, gs3)
        ws = (ws0, ws1, ws2, ws3)

        pltpu.sync_copy(idx_hbm.at[pl.ds(base, _G_ROWS_PER_W)], idx_all)

        def clamp(i, carry):
            sl = pl.ds(i * 16, 16)
            idx_all[sl] = jnp.minimum(jnp.maximum(idx_all[sl], 0),
                                      N_TOKENS - 1)
            return carry

        lax.fori_loop(0, _G_ROWS_PER_W // 16, clamp, 0)

        def g_copy(c, b):
            return pltpu.make_async_copy(
                x_hbm.at[idx_all.at[pl.ds(c * _G_CHUNK, _G_CHUNK)]],
                bufs.at[b], gs[b])

        def w_copy(c, b):
            return pltpu.make_async_copy(
                bufs.at[b],
                out_hbm.at[pl.ds(base + c * _G_CHUNK, _G_CHUNK)], ws[b])

        def grp(g, carry):
            for k in range(_G_NBUF):
                c = g * _G_NBUF + k

                @pl.when(g > 0)
                def _():
                    w_copy(c - _G_NBUF, k).wait()

                g_copy(c, k).start()
                if k == 0:
                    @pl.when(g > 0)
                    def _():
                        g_copy(c - 1, _G_NBUF - 1).wait()
                        w_copy(c - 1, _G_NBUF - 1).start()
                else:
                    g_copy(c - 1, k - 1).wait()
                    w_copy(c - 1, k - 1).start()
            return carry

        lax.fori_loop(0, _G_NGRP, grp, 0)
        last = _G_NCHUNK - 1
        g_copy(last, _G_NBUF - 1).wait()
        w_copy(last, _G_NBUF - 1).start()
        for k in range(_G_NBUF):
            w_copy(0, k).wait()

    return gather_k


def _mlp_block(be_ref, xb_ref, w1_ref, w2_ref, sw_ref, ob_ref):
    del be_ref
    xb = xb_ref[...]
    h = jax.nn.gelu(jnp.dot(xb, w1_ref[0], preferred_element_type=jnp.float32))
    o = jnp.dot(h, w2_ref[0], preferred_element_type=jnp.float32)
    ob_ref[...] = o * sw_ref[0, 0][:, None]


def _grouped_mlp(block_expert, x_perm, w1, w2, slot_w):
    grid_spec = pltpu.PrefetchScalarGridSpec(
        num_scalar_prefetch=1,
        grid=(NB,),
        in_specs=[
            pl.BlockSpec((BLK, D_MODEL), lambda b, be: (b, 0)),
            pl.BlockSpec((1, D_MODEL, D_FF), lambda b, be: (be[b], 0, 0)),
            pl.BlockSpec((1, D_FF, D_MODEL), lambda b, be: (be[b], 0, 0)),
            pl.BlockSpec((1, 1, BLK), lambda b, be: (b, 0, 0)),
        ],
        out_specs=pl.BlockSpec((BLK, D_MODEL), lambda b, be: (b, 0)),
    )
    return pl.pallas_call(
        _mlp_block,
        grid_spec=grid_spec,
        out_shape=jax.ShapeDtypeStruct((PP, D_MODEL), jnp.float32),
    )(block_expert, x_perm, w1, w2, slot_w.reshape(NB, 1, BLK))


_U_TOK = 4
_U_TOK_PER_W = N_TOKENS // NUM_WORKERS
_U_NCHUNK = _U_TOK_PER_W // _U_TOK
_U_NGRP = _U_NCHUNK // 2
_NV = D_MODEL // 16
_U_IDX_PER_W = _U_TOK_PER_W * TOP_K


def _make_unpermute():
    mesh = plsc.VectorSubcoreMesh(core_axis_name="c", subcore_axis_name="s")

    @functools.partial(
        pl.kernel, mesh=mesh,
        out_type=jax.ShapeDtypeStruct((N_TOKENS, D_MODEL), jnp.float32),
        scratch_types=[
            pltpu.VMEM((_U_IDX_PER_W,), jnp.int32),
            pltpu.VMEM((2, _U_TOK * TOP_K, D_MODEL), jnp.float32),
            pltpu.VMEM((2, _U_TOK, D_MODEL), jnp.float32),
            pltpu.SemaphoreType.DMA,
            pltpu.SemaphoreType.DMA,
            pltpu.SemaphoreType.DMA,
            pltpu.SemaphoreType.DMA,
        ],
    )
    def unperm_k(outp_hbm, sidx_hbm, y_hbm, idx_all, rows_v, y_v,
                 gs0, gs1, ys0, ys1):
        wid = lax.axis_index("s") * 2 + lax.axis_index("c")
        tok_base = wid * _U_TOK_PER_W
        gs = (gs0, gs1)
        ys = (ys0, ys1)

        pltpu.sync_copy(sidx_hbm.at[pl.ds(tok_base * TOP_K, _U_IDX_PER_W)],
                        idx_all)

        def g_copy(c, b):
            return pltpu.make_async_copy(
                outp_hbm.at[idx_all.at[pl.ds(c * _U_TOK * TOP_K,
                                             _U_TOK * TOP_K)]],
                rows_v.at[b], gs[b])

        def y_copy(c, b):
            return pltpu.make_async_copy(
                y_v.at[b],
                y_hbm.at[pl.ds(tok_base + c * _U_TOK, _U_TOK)], ys[b])

        g_copy(0, 0).start()

        def grp(g, carry):
            for b in range(2):
                c = g * 2 + b
                g_copy(c, b).wait()

                @pl.when(c + 1 < _U_NCHUNK)
                def _():
                    g_copy(c + 1, 1 - b).start()

                @pl.when(g > 0)
                def _():
                    y_copy(c - 2, b).wait()

                def col_body(v, carry2):
                    sl = pl.ds(v * 16, 16)
                    for j in range(_U_TOK):
                        acc = rows_v[b, j * TOP_K, sl]
                        for r in range(1, TOP_K):
                            acc = acc + rows_v[b, j * TOP_K + r, sl]
                        y_v[b, j, sl] = acc
                    return carry2

                lax.fori_loop(0, _NV, col_body, 0)
                y_copy(c, b).start()
            return carry

        lax.fori_loop(0, _U_NGRP, grp, 0)
        y_copy(0, 0).wait()
        y_copy(0, 1).wait()

    return unperm_k


def kernel(x, expert_weights, expert_indices, batch_size_per_expert, w1, w2):
    block_expert, dest = _routing_tables(expert_indices,
                                         batch_size_per_expert)
    flat_w = expert_weights.reshape(-1, _S_CHUNK).astype(jnp.float32)
    tok_of_assign = (jnp.arange(P, dtype=jnp.int32) // TOP_K).reshape(
        -1, _S_CHUNK)

    src_tok, slot_w = _make_scatter_meta()(dest.reshape(-1, _S_CHUNK),
                                           tok_of_assign, flat_w)
    x_perm = _make_gather()(x, src_tok)
    out_perm = _grouped_mlp(block_expert, x_perm, w1, w2, slot_w)
    y = _make_unpermute()(out_perm, dest)
    return y
